# baseline (speedup 1.0000x reference)
import jax
import jax.numpy as jnp
from jax.experimental import pallas as pl
from jax.experimental.pallas import tpu as pltpu


def _round_up(n, m):
    return ((n + m - 1) // m) * m


def _mlp_packed_body(x_ref, w1_ref, b1_ref, w2_ref, b2_ref, o_ref):
    h = jnp.dot(x_ref[...], w1_ref[...], preferred_element_type=jnp.float32)
    h = jnp.maximum(h + b1_ref[...], 0.0)
    out = jnp.dot(h, w2_ref[...], preferred_element_type=jnp.float32)
    o_ref[...] = (out + b2_ref[...]).astype(o_ref.dtype)


def _block_diag(w, p):
    d, h = w.shape
    eye = jnp.eye(p, dtype=w.dtype)
    return (eye[:, None, :, None] * w[None, :, None, :]).reshape(p * d, p * h)


def kernel(x, w1p, b1p, w2p, b2p):
    B, D = x.shape
    Hp = w1p.shape[1]
    O = w2p.shape[1]
    f32 = jnp.float32
    x = x.astype(f32)

    P = 1
    while P * 2 * D <= 256 and B % (P * 2) == 0:
        P *= 2

    xr = x.reshape(B // P, P * D)
    w1b = _block_diag(w1p.astype(f32), P)
    w2b = _block_diag(w2p.astype(f32), P)
    b1b = jnp.tile(b1p.astype(f32), (1, P))
    b2b = jnp.tile(b2p.astype(f32), (1, P))

    Bp = B // P
    block_m = min(2048, max(_round_up(-(-Bp // 2), 8), 8))
    Bpp = _round_up(Bp, block_m)
    if Bpp != Bp:
        xr = jnp.zeros((Bpp, P * D), f32).at[:Bp].set(xr)

    out_p = pl.pallas_call(
        _mlp_packed_body,
        out_shape=jax.ShapeDtypeStruct((Bpp, P * O), f32),
        grid_spec=pl.GridSpec(
            grid=(Bpp // block_m,),
            in_specs=[
                pl.BlockSpec((block_m, P * D), lambda i: (i, 0)),
                pl.BlockSpec((P * D, P * Hp), lambda i: (0, 0)),
                pl.BlockSpec((1, P * Hp), lambda i: (0, 0)),
                pl.BlockSpec((P * Hp, P * O), lambda i: (0, 0)),
                pl.BlockSpec((1, P * O), lambda i: (0, 0)),
            ],
            out_specs=pl.BlockSpec((block_m, P * O), lambda i: (i, 0)),
        ),
        compiler_params=pltpu.CompilerParams(
            dimension_semantics=("parallel",)),
    )(xr, w1b, b1b, w2b, b2b)

    return out_p[:Bp].reshape(B, O)


# speedup vs baseline: 6.2835x; 6.2835x over previous
import jax
import jax.numpy as jnp
from jax.experimental import pallas as pl
from jax.experimental.pallas import tpu as pltpu


def _mlp_t_body(x_ref, w1t_ref, b1t_ref, w2t_ref, b2t_ref, o_ref):
    h = jnp.dot(w1t_ref[...], x_ref[...],
                preferred_element_type=jnp.float32)
    h = jnp.maximum(h + b1t_ref[...], 0.0)
    out = jnp.dot(w2t_ref[...], h,
                  preferred_element_type=jnp.float32)
    o_ref[...] = (out + b2t_ref[...]).astype(o_ref.dtype)


def kernel(x, w1p, b1p, w2p, b2p):
    B, D = x.shape
    Hp = w1p.shape[1]
    O = w2p.shape[1]
    f32 = jnp.float32

    xt = x.astype(f32).T
    w1t = w1p.astype(f32).T
    w2t = w2p.astype(f32).T
    b1t = b1p.astype(f32).T
    b2t = b2p.astype(f32).T

    block_n = 8192
    while block_n > 128 and B % block_n != 0:
        block_n //= 2
    grid_n = B // block_n

    out_t = pl.pallas_call(
        _mlp_t_body,
        out_shape=jax.ShapeDtypeStruct((O, B), f32),
        grid_spec=pl.GridSpec(
            grid=(grid_n,),
            in_specs=[
                pl.BlockSpec((D, block_n), lambda i: (0, i)),
                pl.BlockSpec((Hp, D), lambda i: (0, 0)),
                pl.BlockSpec((Hp, 1), lambda i: (0, 0)),
                pl.BlockSpec((O, Hp), lambda i: (0, 0)),
                pl.BlockSpec((O, 1), lambda i: (0, 0)),
            ],
            out_specs=pl.BlockSpec((O, block_n), lambda i: (0, i)),
        ),
        compiler_params=pltpu.CompilerParams(
            dimension_semantics=("parallel",)),
    )(xt, w1t, b1t, w2t, b2t)

    return out_t.T
